# baseline (speedup 1.0000x reference)
import functools

import jax
import jax.numpy as jnp
from jax import lax
from jax.experimental import pallas as pl
from jax.experimental.pallas import tpu as pltpu
from jax.experimental.pallas import tpu_sc as plsc

N = 10000
D = 128
OUT = 32
E = 320000
EPS = 1e-5

NPAD = 10240
EPAD = 327680
CH = 64
EPT = EPAD // 32
NCH = EPT // CH
RPT = NPAD // 16
BLK = 1024


def _s2_body(u_hbm, src_hbm, dst_hbm, out_hbm, sidx, didx_a, didx_b,
             rows_a, rows_b, zbuf, acc3, gs_a, gs_b, is_a, is_b, ss_a, ss_b):
    c = lax.axis_index("c")
    s = lax.axis_index("s")
    wid = s * 2 + c
    base = wid * EPT

    didx = (didx_a, didx_b)
    rows = (rows_a, rows_b)
    gs = (gs_a, gs_b)
    isem = (is_a, is_b)
    ss = (ss_a, ss_b)

    pltpu.sync_copy(src_hbm.at[pl.ds(base, EPT)], sidx)

    zero16 = jnp.zeros((16,), jnp.float32)
    for i in range(CH):
        for j in range(D // 16):
            zbuf[i, pl.ds(j * 16, 16)] = zero16
    for k in range(RPT // CH):
        pltpu.sync_copy(zbuf, out_hbm.at[c, pl.ds(s * RPT + k * CH, CH)])
    plsc.subcore_barrier()

    def start_pair(ci, x):
        pltpu.async_copy(dst_hbm.at[pl.ds(base + ci * CH, CH)], didx[x],
                         isem[x])
        pltpu.async_copy(u_hbm.at[sidx.at[pl.ds(ci * CH, CH)]], rows[x],
                         gs[x])

    def wait_dma(hbm_src, dst, sem):
        pltpu.make_async_copy(hbm_src, dst, sem).wait()

    def step(ci, x, y):
        @pl.when(ci + 1 < NCH)
        def _():
            @pl.when(ci >= 1)
            def _():
                wait_dma(u_hbm.at[pl.ds(0, CH)], rows[y], ss[y])
            start_pair(ci + 1, y)

        wait_dma(dst_hbm.at[pl.ds(0, CH)], didx[x], isem[x])
        wait_dma(u_hbm.at[pl.ds(0, CH)], rows[x], gs[x])
        pltpu.async_copy(rows[x], acc3.at[pl.ds(s * 160, CH)], ss[x])

    start_pair(0, 0)

    def pair(mj, carry):
        step(2 * mj, 0, 1)
        step(2 * mj + 1, 1, 0)
        return carry

    lax.fori_loop(0, NCH // 2, pair, 0)
    wait_dma(u_hbm.at[pl.ds(0, CH)], rows[0], ss[0])
    wait_dma(u_hbm.at[pl.ds(0, CH)], rows[1], ss[1])
    plsc.subcore_barrier()


@functools.cache
def _get_s2():
    return pl.kernel(
        _s2_body,
        out_type=jax.ShapeDtypeStruct((2, NPAD, D), jnp.float32),
        mesh=plsc.VectorSubcoreMesh(core_axis_name="c", subcore_axis_name="s"),
        scratch_types=[
            pltpu.VMEM((EPT,), jnp.int32),
            pltpu.VMEM((CH,), jnp.int32),
            pltpu.VMEM((CH,), jnp.int32),
            pltpu.VMEM((CH, 3 * D), jnp.float32),
            pltpu.VMEM((CH, 3 * D), jnp.float32),
            pltpu.VMEM((CH, D), jnp.float32),
            pltpu.VMEM_SHARED((2560, 3 * D), jnp.float32),
            pltpu.SemaphoreType.DMA,
            pltpu.SemaphoreType.DMA,
            pltpu.SemaphoreType.DMA,
            pltpu.SemaphoreType.DMA,
            pltpu.SemaphoreType.DMA,
            pltpu.SemaphoreType.DMA,
        ],
    )


def _s2(u, srcp, dstp):
    u3 = jnp.concatenate([u, u, u], axis=1)
    return _get_s2()(u3, srcp, dstp)


def _dis_body(da, db, dis_ref, dsq_ref):
    d = da[...] + db[...]
    row = lax.broadcasted_iota(jnp.int32, (NPAD, D), 0)
    ok = (d > 0) & (row < N)
    r = lax.rsqrt(jnp.where(ok, d, 1.0))
    dis = jnp.where(ok, r, 0.0)
    dis_ref[...] = dis
    dsq_ref[...] = dis * dis


_k_dis = pl.pallas_call(
    _dis_body,
    out_shape=(jax.ShapeDtypeStruct((NPAD, D), jnp.float32),
               jax.ShapeDtypeStruct((NPAD, D), jnp.float32)),
)


def _scale_body(a, b, o_ref):
    o_ref[...] = a[...] * b[...]


_k_scale = pl.pallas_call(
    _scale_body,
    out_shape=jax.ShapeDtypeStruct((NPAD, D), jnp.float32),
)


def _c1_body(a, b, dsq, o_ref):
    o_ref[...] = -dsq[...] * (a[...] + b[...])


_k_c1 = pl.pallas_call(
    _c1_body,
    out_shape=jax.ShapeDtypeStruct((NPAD, D), jnp.float32),
)


def _c2_body(a, b, dsq, u0, o_ref):
    o_ref[...] = -2.0 * dsq[...] * (a[...] + b[...]) - u0[...]


_k_c2 = pl.pallas_call(
    _c2_body,
    out_shape=jax.ShapeDtypeStruct((NPAD, D), jnp.float32),
)


def _pre_body(h, a1, b1, a2, b2, a3, b3, dis, W, bias, pre_ref, st_ref):
    i = pl.program_id(0)
    A = W[0] - W[2]
    B1 = W[3] - W[1]
    s1 = a1[...] + b1[...]
    s2 = a2[...] + b2[...]
    s3 = a3[...] + b3[...]
    acc = jnp.dot(h[...], A, preferred_element_type=jnp.float32)
    m = (jnp.dot(s1, B1, preferred_element_type=jnp.float32)
         - 2.0 * jnp.dot(s2, W[2], preferred_element_type=jnp.float32)
         - 2.0 * jnp.dot(s3, W[3], preferred_element_type=jnp.float32))
    pre = acc + dis[...] * m + bias[...]
    row = i * BLK + lax.broadcasted_iota(jnp.int32, (BLK, D), 0)
    pre = jnp.where(row < N, pre, 0.0)
    pre_ref[...] = pre

    @pl.when(i == 0)
    def _():
        st_ref[...] = jnp.zeros_like(st_ref)

    st_ref[0:1, :] += jnp.sum(pre, axis=0, keepdims=True)
    st_ref[1:2, :] += jnp.sum(pre * pre, axis=0, keepdims=True)


_row_spec = pl.BlockSpec((BLK, D), lambda i: (i, 0))

_k_pre = pl.pallas_call(
    _pre_body,
    grid=(NPAD // BLK,),
    in_specs=[_row_spec] * 8 + [
        pl.BlockSpec((4, D, D), lambda i: (0, 0, 0)),
        pl.BlockSpec((1, D), lambda i: (0, 0)),
    ],
    out_specs=(_row_spec, pl.BlockSpec((2, D), lambda i: (0, 0))),
    out_shape=(jax.ShapeDtypeStruct((NPAD, D), jnp.float32),
               jax.ShapeDtypeStruct((2, D), jnp.float32)),
)


def _gn(pre, st, gw, gb, gms):
    mean = st[0:1, :] / N
    ex2 = st[1:2, :] / N
    ms = gms[...]
    var = ex2 - 2.0 * ms * mean * mean + ms * ms * mean * mean
    return gw[...] * (pre[...] - ms * mean) * lax.rsqrt(var + EPS) + gb[...]


def _apply_mid_body(pre, st, dis, gw, gb, gms, h_ref, u0_ref):
    g = _gn(pre, st, gw, gb, gms)
    h = jnp.where(g >= 0, g, 0.1 * g)
    row = lax.broadcasted_iota(jnp.int32, (NPAD, D), 0)
    h = jnp.where(row < N, h, 0.0)
    h_ref[...] = h
    u0_ref[...] = dis[...] * h


_k_apply_mid = pl.pallas_call(
    _apply_mid_body,
    out_shape=(jax.ShapeDtypeStruct((NPAD, D), jnp.float32),
               jax.ShapeDtypeStruct((NPAD, D), jnp.float32)),
)


def _apply_last_body(pre, st, feat, gw, gb, gms, lw, lb, o_ref):
    g = _gn(pre, st, gw, gb, gms)
    h = jnp.maximum(feat[...] + g, 0.0)
    row = lax.broadcasted_iota(jnp.int32, (NPAD, D), 0)
    h = jnp.where(row < N, h, 0.0)
    pooled = jnp.maximum(jnp.sum(h, axis=0, keepdims=True) / N, 0.0)
    logits = jnp.dot(pooled, lw[...], preferred_element_type=jnp.float32)
    logits = logits + lb[...]
    mx = jnp.max(logits, axis=-1, keepdims=True)
    e = jnp.exp(logits - mx)
    o_ref[...] = e / jnp.sum(e, axis=-1, keepdims=True)


_k_apply_last = pl.pallas_call(
    _apply_last_body,
    out_shape=jax.ShapeDtypeStruct((1, OUT), jnp.float32),
)


def _branch(featp, srcp, dstp, dis, dsq, W, bc, gw, gb, gms, lw, lb):
    h = featp
    u0 = _k_scale(dis, featp)
    for i in range(4):
        a1 = _s2(u0, srcp, dstp)
        u1 = _k_c1(a1[0], a1[1], dsq)
        a2 = _s2(u1, srcp, dstp)
        u2 = _k_c2(a2[0], a2[1], dsq, u0)
        a3 = _s2(u2, srcp, dstp)
        pre, st = _k_pre(h, a1[0], a1[1], a2[0], a2[1], a3[0], a3[1], dis,
                         W[i], bc[i][None, :])
        if i < 3:
            h, u0 = _k_apply_mid(pre, st, dis, gw[i][None, :], gb[i][None, :],
                                 gms[i][None, :])
        else:
            out = _k_apply_last(pre, st, featp, gw[3][None, :],
                                gb[3][None, :], gms[3][None, :], lw,
                                lb[None, :])
    return out[0]


def kernel(edge_index, feat, feat_1, feat_2, W1, b1, gn1_w, gn1_b, gn1_ms,
           lin1_w, lin1_b, W2, b2, gn2_w, gn2_b, gn2_ms, lin2_w, lin2_b,
           W3, b3, gn3_w, gn3_b, gn3_ms, lin3_w, lin3_b):
    src = edge_index[0]
    dst = edge_index[1]
    pad = jnp.full((EPAD - E,), NPAD - 1, jnp.int32)
    srcp = jnp.concatenate([src, pad])
    dstp = jnp.concatenate([dst, pad])
    f1 = jnp.pad(feat, ((0, NPAD - N), (0, 0)))
    f2 = jnp.pad(feat_1, ((0, NPAD - N), (0, 0)))
    f3 = jnp.pad(feat_2, ((0, NPAD - N), (0, 0)))
    ones = jnp.ones((NPAD, D), jnp.float32)

    deg2 = _s2(ones, srcp, srcp)
    dis, dsq = _k_dis(deg2[0], deg2[1])

    out1 = _branch(f1, srcp, dstp, dis, dsq, W1, b1, gn1_w, gn1_b, gn1_ms,
                   lin1_w, lin1_b)
    out2 = _branch(f2, srcp, dstp, dis, dsq, W2, b2, gn2_w, gn2_b, gn2_ms,
                   lin2_w, lin2_b)
    out3 = _branch(f3, srcp, dstp, dis, dsq, W3, b3, gn3_w, gn3_b, gn3_ms,
                   lin3_w, lin3_b)
    return (out1, out2, out3)


# speedup vs baseline: 1.3174x; 1.3174x over previous
import functools

import jax
import jax.numpy as jnp
from jax import lax
from jax.experimental import pallas as pl
from jax.experimental.pallas import tpu as pltpu
from jax.experimental.pallas import tpu_sc as plsc

N = 10000
D = 128
OUT = 32
E = 320000
EPS = 1e-5

NPAD = 10240
EPAD = 327680
CH = 128
EPT = EPAD // 32
NCH = EPT // CH
RPT = NPAD // 16
BLK = 1024


def _s2_body(u_hbm, src_hbm, dst_hbm, out_hbm, sidx, didx_a, didx_b,
             rows_a, rows_b, acc, gs_a, gs_b, is_a, is_b, ss_a, ss_b):
    c = lax.axis_index("c")
    s = lax.axis_index("s")
    wid = s * 2 + c
    base = wid * EPT

    didx = (didx_a, didx_b)
    rows = (rows_a, rows_b)
    gs = (gs_a, gs_b)
    isem = (is_a, is_b)
    ss = (ss_a, ss_b)

    pltpu.sync_copy(src_hbm.at[pl.ds(base, EPT)], sidx)

    zero16 = jnp.zeros((16,), jnp.float32)

    def zrow(i, carry):
        for j in range(D // 16):
            rows_a[i, pl.ds(j * 16, 16)] = zero16
        return carry

    lax.fori_loop(0, CH, zrow, 0)
    for k in range(RPT // CH):
        pltpu.sync_copy(rows_a, acc.at[pl.ds(s * RPT + k * CH, CH)])
    plsc.subcore_barrier()

    def start_pair(ci, x):
        pltpu.async_copy(dst_hbm.at[pl.ds(base + ci * CH, CH)], didx[x],
                         isem[x])
        pltpu.async_copy(u_hbm.at[sidx.at[pl.ds(ci * CH, CH)]], rows[x],
                         gs[x])

    def wait_dma(hbm_src, dst, sem):
        pltpu.make_async_copy(hbm_src, dst, sem).wait()

    def step(ci, x, y):
        @pl.when(ci + 1 < NCH)
        def _():
            @pl.when(ci >= 1)
            def _():
                wait_dma(u_hbm.at[pl.ds(0, CH)], rows[y], ss[y])
            start_pair(ci + 1, y)

        wait_dma(dst_hbm.at[pl.ds(0, CH)], didx[x], isem[x])
        wait_dma(u_hbm.at[pl.ds(0, CH)], rows[x], gs[x])
        pltpu.async_copy(rows[x], acc.at[didx[x]], ss[x], add=True)

    start_pair(0, 0)

    def pair(mj, carry):
        step(2 * mj, 0, 1)
        step(2 * mj + 1, 1, 0)
        return carry

    lax.fori_loop(0, NCH // 2, pair, 0)
    wait_dma(u_hbm.at[pl.ds(0, CH)], rows[0], ss[0])
    wait_dma(u_hbm.at[pl.ds(0, CH)], rows[1], ss[1])
    plsc.subcore_barrier()

    for k in range(RPT // CH):
        r0 = s * RPT + k * CH
        pltpu.sync_copy(acc.at[pl.ds(r0, CH)], out_hbm.at[c, pl.ds(r0, CH)])


@functools.cache
def _get_s2():
    return pl.kernel(
        _s2_body,
        out_type=jax.ShapeDtypeStruct((2, NPAD, D), jnp.float32),
        mesh=plsc.VectorSubcoreMesh(core_axis_name="c", subcore_axis_name="s"),
        scratch_types=[
            pltpu.VMEM((EPT,), jnp.int32),
            pltpu.VMEM((CH,), jnp.int32),
            pltpu.VMEM((CH,), jnp.int32),
            pltpu.VMEM((CH, D), jnp.float32),
            pltpu.VMEM((CH, D), jnp.float32),
            pltpu.VMEM_SHARED((NPAD, D), jnp.float32),
            pltpu.SemaphoreType.DMA,
            pltpu.SemaphoreType.DMA,
            pltpu.SemaphoreType.DMA,
            pltpu.SemaphoreType.DMA,
            pltpu.SemaphoreType.DMA,
            pltpu.SemaphoreType.DMA,
        ],
    )


def _s2(u, srcp, dstp):
    return _get_s2()(u, srcp, dstp)


def _dis_body(da, db, dis_ref, dsq_ref):
    d = da[...] + db[...]
    row = lax.broadcasted_iota(jnp.int32, (NPAD, D), 0)
    ok = (d > 0) & (row < N)
    r = lax.rsqrt(jnp.where(ok, d, 1.0))
    dis = jnp.where(ok, r, 0.0)
    dis_ref[...] = dis
    dsq_ref[...] = dis * dis


_k_dis = pl.pallas_call(
    _dis_body,
    out_shape=(jax.ShapeDtypeStruct((NPAD, D), jnp.float32),
               jax.ShapeDtypeStruct((NPAD, D), jnp.float32)),
)


def _scale_body(a, b, o_ref):
    o_ref[...] = a[...] * b[...]


_k_scale = pl.pallas_call(
    _scale_body,
    out_shape=jax.ShapeDtypeStruct((NPAD, D), jnp.float32),
)


def _c1_body(a, b, dsq, o_ref):
    o_ref[...] = -dsq[...] * (a[...] + b[...])


_k_c1 = pl.pallas_call(
    _c1_body,
    out_shape=jax.ShapeDtypeStruct((NPAD, D), jnp.float32),
)


def _c2_body(a, b, dsq, u0, o_ref):
    o_ref[...] = -2.0 * dsq[...] * (a[...] + b[...]) - u0[...]


_k_c2 = pl.pallas_call(
    _c2_body,
    out_shape=jax.ShapeDtypeStruct((NPAD, D), jnp.float32),
)


def _pre_body(h, a1, b1, a2, b2, a3, b3, dis, W, bias, pre_ref, st_ref):
    i = pl.program_id(0)
    A = W[0] - W[2]
    B1 = W[3] - W[1]
    s1 = a1[...] + b1[...]
    s2 = a2[...] + b2[...]
    s3 = a3[...] + b3[...]
    acc = jnp.dot(h[...], A, preferred_element_type=jnp.float32)
    m = (jnp.dot(s1, B1, preferred_element_type=jnp.float32)
         - 2.0 * jnp.dot(s2, W[2], preferred_element_type=jnp.float32)
         - 2.0 * jnp.dot(s3, W[3], preferred_element_type=jnp.float32))
    pre = acc + dis[...] * m + bias[...]
    row = i * BLK + lax.broadcasted_iota(jnp.int32, (BLK, D), 0)
    pre = jnp.where(row < N, pre, 0.0)
    pre_ref[...] = pre

    @pl.when(i == 0)
    def _():
        st_ref[...] = jnp.zeros_like(st_ref)

    st_ref[0:1, :] += jnp.sum(pre, axis=0, keepdims=True)
    st_ref[1:2, :] += jnp.sum(pre * pre, axis=0, keepdims=True)


_row_spec = pl.BlockSpec((BLK, D), lambda i: (i, 0))

_k_pre = pl.pallas_call(
    _pre_body,
    grid=(NPAD // BLK,),
    in_specs=[_row_spec] * 8 + [
        pl.BlockSpec((4, D, D), lambda i: (0, 0, 0)),
        pl.BlockSpec((1, D), lambda i: (0, 0)),
    ],
    out_specs=(_row_spec, pl.BlockSpec((2, D), lambda i: (0, 0))),
    out_shape=(jax.ShapeDtypeStruct((NPAD, D), jnp.float32),
               jax.ShapeDtypeStruct((2, D), jnp.float32)),
)


def _gn(pre, st, gw, gb, gms):
    mean = st[0:1, :] / N
    ex2 = st[1:2, :] / N
    ms = gms[...]
    var = ex2 - 2.0 * ms * mean * mean + ms * ms * mean * mean
    return gw[...] * (pre[...] - ms * mean) * lax.rsqrt(var + EPS) + gb[...]


def _apply_mid_body(pre, st, dis, gw, gb, gms, h_ref, u0_ref):
    g = _gn(pre, st, gw, gb, gms)
    h = jnp.where(g >= 0, g, 0.1 * g)
    row = lax.broadcasted_iota(jnp.int32, (NPAD, D), 0)
    h = jnp.where(row < N, h, 0.0)
    h_ref[...] = h
    u0_ref[...] = dis[...] * h


_k_apply_mid = pl.pallas_call(
    _apply_mid_body,
    out_shape=(jax.ShapeDtypeStruct((NPAD, D), jnp.float32),
               jax.ShapeDtypeStruct((NPAD, D), jnp.float32)),
)


def _apply_last_body(pre, st, feat, gw, gb, gms, lw, lb, o_ref):
    g = _gn(pre, st, gw, gb, gms)
    h = jnp.maximum(feat[...] + g, 0.0)
    row = lax.broadcasted_iota(jnp.int32, (NPAD, D), 0)
    h = jnp.where(row < N, h, 0.0)
    pooled = jnp.maximum(jnp.sum(h, axis=0, keepdims=True) / N, 0.0)
    logits = jnp.dot(pooled, lw[...], preferred_element_type=jnp.float32)
    logits = logits + lb[...]
    mx = jnp.max(logits, axis=-1, keepdims=True)
    e = jnp.exp(logits - mx)
    o_ref[...] = e / jnp.sum(e, axis=-1, keepdims=True)


_k_apply_last = pl.pallas_call(
    _apply_last_body,
    out_shape=jax.ShapeDtypeStruct((1, OUT), jnp.float32),
)


def _branch(featp, srcp, dstp, dis, dsq, W, bc, gw, gb, gms, lw, lb):
    h = featp
    u0 = _k_scale(dis, featp)
    for i in range(4):
        a1 = _s2(u0, srcp, dstp)
        u1 = _k_c1(a1[0], a1[1], dsq)
        a2 = _s2(u1, srcp, dstp)
        u2 = _k_c2(a2[0], a2[1], dsq, u0)
        a3 = _s2(u2, srcp, dstp)
        pre, st = _k_pre(h, a1[0], a1[1], a2[0], a2[1], a3[0], a3[1], dis,
                         W[i], bc[i][None, :])
        if i < 3:
            h, u0 = _k_apply_mid(pre, st, dis, gw[i][None, :], gb[i][None, :],
                                 gms[i][None, :])
        else:
            out = _k_apply_last(pre, st, featp, gw[3][None, :],
                                gb[3][None, :], gms[3][None, :], lw,
                                lb[None, :])
    return out[0]


def kernel(edge_index, feat, feat_1, feat_2, W1, b1, gn1_w, gn1_b, gn1_ms,
           lin1_w, lin1_b, W2, b2, gn2_w, gn2_b, gn2_ms, lin2_w, lin2_b,
           W3, b3, gn3_w, gn3_b, gn3_ms, lin3_w, lin3_b):
    src = edge_index[0]
    dst = edge_index[1]
    pad = jnp.full((EPAD - E,), NPAD - 1, jnp.int32)
    srcp = jnp.concatenate([src, pad])
    dstp = jnp.concatenate([dst, pad])
    f1 = jnp.pad(feat, ((0, NPAD - N), (0, 0)))
    f2 = jnp.pad(feat_1, ((0, NPAD - N), (0, 0)))
    f3 = jnp.pad(feat_2, ((0, NPAD - N), (0, 0)))
    ones = jnp.ones((NPAD, D), jnp.float32)

    deg2 = _s2(ones, srcp, srcp)
    dis, dsq = _k_dis(deg2[0], deg2[1])

    out1 = _branch(f1, srcp, dstp, dis, dsq, W1, b1, gn1_w, gn1_b, gn1_ms,
                   lin1_w, lin1_b)
    out2 = _branch(f2, srcp, dstp, dis, dsq, W2, b2, gn2_w, gn2_b, gn2_ms,
                   lin2_w, lin2_b)
    out3 = _branch(f3, srcp, dstp, dis, dsq, W3, b3, gn3_w, gn3_b, gn3_ms,
                   lin3_w, lin3_b)
    return (out1, out2, out3)


# speedup vs baseline: 1.3727x; 1.0420x over previous
import functools

import jax
import jax.numpy as jnp
from jax import lax
from jax.experimental import pallas as pl
from jax.experimental.pallas import tpu as pltpu
from jax.experimental.pallas import tpu_sc as plsc

N = 10000
D = 128
OUT = 32
E = 320000
EPS = 1e-5

NPAD = 10240
EPAD = 327680
CH = 128
EPT = EPAD // 32
NCH = EPT // CH
RPT = NPAD // 16
BLK = 1024


def _s2_body(u_hbm, src_hbm, dst_hbm, out_hbm, sidx, didx_a, didx_b,
             rows_a, rows_b, acc, gs_a, gs_b, is_a, is_b, ss_a, ss_b):
    c = lax.axis_index("c")
    s = lax.axis_index("s")
    wid = s * 2 + c
    base = wid * EPT

    didx = (didx_a, didx_b)
    rows = (rows_a, rows_b)
    gs = (gs_a, gs_b)
    isem = (is_a, is_b)
    ss = (ss_a, ss_b)

    pltpu.sync_copy(src_hbm.at[pl.ds(base, EPT)], sidx)

    zero16 = jnp.zeros((16,), jnp.float32)
    for i in range(CH):
        for j in range(D // 16):
            rows_a[i, pl.ds(j * 16, 16)] = zero16
    for k in range(RPT // CH):
        pltpu.sync_copy(rows_a, acc.at[pl.ds(s * RPT + k * CH, CH)])
    plsc.subcore_barrier()

    def start_pair(ci, x):
        pltpu.async_copy(dst_hbm.at[pl.ds(base + ci * CH, CH)], didx[x],
                         isem[x])
        pltpu.async_copy(u_hbm.at[sidx.at[pl.ds(ci * CH, CH)]], rows[x],
                         gs[x])

    def wait_dma(hbm_src, dst, sem):
        pltpu.make_async_copy(hbm_src, dst, sem).wait()

    def step(ci, x, y):
        @pl.when(ci + 1 < NCH)
        def _():
            @pl.when(ci >= 1)
            def _():
                wait_dma(u_hbm.at[pl.ds(0, CH)], rows[y], ss[y])
            start_pair(ci + 1, y)

        wait_dma(dst_hbm.at[pl.ds(0, CH)], didx[x], isem[x])
        wait_dma(u_hbm.at[pl.ds(0, CH)], rows[x], gs[x])
        pltpu.async_copy(rows[x], acc.at[didx[x]], ss[x], add=True)

    start_pair(0, 0)

    def pair(mj, carry):
        step(2 * mj, 0, 1)
        step(2 * mj + 1, 1, 0)
        return carry

    lax.fori_loop(0, NCH // 2, pair, 0)
    wait_dma(u_hbm.at[pl.ds(0, CH)], rows[0], ss[0])
    wait_dma(u_hbm.at[pl.ds(0, CH)], rows[1], ss[1])
    plsc.subcore_barrier()

    for k in range(RPT // CH):
        r0 = s * RPT + k * CH
        pltpu.sync_copy(acc.at[pl.ds(r0, CH)], out_hbm.at[c, pl.ds(r0, CH)])


@functools.cache
def _get_s2():
    return pl.kernel(
        _s2_body,
        out_type=jax.ShapeDtypeStruct((2, NPAD, D), jnp.float32),
        mesh=plsc.VectorSubcoreMesh(core_axis_name="c", subcore_axis_name="s"),
        scratch_types=[
            pltpu.VMEM((EPT,), jnp.int32),
            pltpu.VMEM((CH,), jnp.int32),
            pltpu.VMEM((CH,), jnp.int32),
            pltpu.VMEM((CH, D), jnp.float32),
            pltpu.VMEM((CH, D), jnp.float32),
            pltpu.VMEM_SHARED((NPAD, D), jnp.float32),
            pltpu.SemaphoreType.DMA,
            pltpu.SemaphoreType.DMA,
            pltpu.SemaphoreType.DMA,
            pltpu.SemaphoreType.DMA,
            pltpu.SemaphoreType.DMA,
            pltpu.SemaphoreType.DMA,
        ],
    )


def _sdeg_body(dst_hbm, out_hbm, didx_a, didx_b, rows_a, acc,
               is_a, is_b, ss_a, ss_b):
    c = lax.axis_index("c")
    s = lax.axis_index("s")
    wid = s * 2 + c
    base = wid * EPT

    didx = (didx_a, didx_b)
    isem = (is_a, is_b)
    ss = (ss_a, ss_b)

    zero16 = jnp.zeros((16,), jnp.float32)
    for i in range(CH):
        for j in range(D // 16):
            rows_a[i, pl.ds(j * 16, 16)] = zero16
    for k in range(RPT // CH):
        pltpu.sync_copy(rows_a, acc.at[pl.ds(s * RPT + k * CH, CH)])
    plsc.subcore_barrier()

    one16 = jnp.ones((16,), jnp.float32)
    for i in range(CH):
        for j in range(D // 16):
            rows_a[i, pl.ds(j * 16, 16)] = one16

    def wait_dma(hbm_src, dst, sem):
        pltpu.make_async_copy(hbm_src, dst, sem).wait()

    def step(ci, x, y):
        @pl.when(ci + 1 < NCH)
        def _():
            @pl.when(ci >= 1)
            def _():
                wait_dma(out_hbm.at[0, pl.ds(0, CH)], rows_a, ss[y])
            pltpu.async_copy(dst_hbm.at[pl.ds(base + (ci + 1) * CH, CH)],
                             didx[y], isem[y])

        wait_dma(dst_hbm.at[pl.ds(0, CH)], didx[x], isem[x])
        pltpu.async_copy(rows_a, acc.at[didx[x]], ss[x], add=True)

    pltpu.async_copy(dst_hbm.at[pl.ds(base, CH)], didx[0], isem[0])

    def pair(mj, carry):
        step(2 * mj, 0, 1)
        step(2 * mj + 1, 1, 0)
        return carry

    lax.fori_loop(0, NCH // 2, pair, 0)
    wait_dma(out_hbm.at[0, pl.ds(0, CH)], rows_a, ss[0])
    wait_dma(out_hbm.at[0, pl.ds(0, CH)], rows_a, ss[1])
    plsc.subcore_barrier()

    for k in range(RPT // CH):
        r0 = s * RPT + k * CH
        pltpu.sync_copy(acc.at[pl.ds(r0, CH)], out_hbm.at[c, pl.ds(r0, CH)])


@functools.cache
def _get_sdeg():
    return pl.kernel(
        _sdeg_body,
        out_type=jax.ShapeDtypeStruct((2, NPAD, D), jnp.float32),
        mesh=plsc.VectorSubcoreMesh(core_axis_name="c", subcore_axis_name="s"),
        scratch_types=[
            pltpu.VMEM((CH,), jnp.int32),
            pltpu.VMEM((CH,), jnp.int32),
            pltpu.VMEM((CH, D), jnp.float32),
            pltpu.VMEM_SHARED((NPAD, D), jnp.float32),
            pltpu.SemaphoreType.DMA,
            pltpu.SemaphoreType.DMA,
            pltpu.SemaphoreType.DMA,
            pltpu.SemaphoreType.DMA,
        ],
    )


def _s2(u, srcp, dstp):
    return _get_s2()(u, srcp, dstp)


def _sdeg(dstp):
    return _get_sdeg()(dstp)


def _dis_body(da, db, dis_ref, dsq_ref):
    d = da[...] + db[...]
    row = lax.broadcasted_iota(jnp.int32, (NPAD, D), 0)
    ok = (d > 0) & (row < N)
    r = lax.rsqrt(jnp.where(ok, d, 1.0))
    dis = jnp.where(ok, r, 0.0)
    dis_ref[...] = dis
    dsq_ref[...] = dis * dis


_k_dis = pl.pallas_call(
    _dis_body,
    out_shape=(jax.ShapeDtypeStruct((NPAD, D), jnp.float32),
               jax.ShapeDtypeStruct((NPAD, D), jnp.float32)),
)


def _scale_body(a, b, o_ref):
    o_ref[...] = a[...] * b[...]


_k_scale = pl.pallas_call(
    _scale_body,
    out_shape=jax.ShapeDtypeStruct((NPAD, D), jnp.float32),
)


def _c1_body(a, b, dsq, o_ref):
    o_ref[...] = -dsq[...] * (a[...] + b[...])


_k_c1 = pl.pallas_call(
    _c1_body,
    out_shape=jax.ShapeDtypeStruct((NPAD, D), jnp.float32),
)


def _c2_body(a, b, dsq, u0, o_ref):
    o_ref[...] = -2.0 * dsq[...] * (a[...] + b[...]) - u0[...]


_k_c2 = pl.pallas_call(
    _c2_body,
    out_shape=jax.ShapeDtypeStruct((NPAD, D), jnp.float32),
)


def _pre_body(h, a1, b1, a2, b2, a3, b3, dis, W, bias, pre_ref, st_ref):
    i = pl.program_id(0)
    A = W[0] - W[2]
    B1 = W[3] - W[1]
    s1 = a1[...] + b1[...]
    s2 = a2[...] + b2[...]
    s3 = a3[...] + b3[...]
    acc = jnp.dot(h[...], A, preferred_element_type=jnp.float32)
    m = (jnp.dot(s1, B1, preferred_element_type=jnp.float32)
         - 2.0 * jnp.dot(s2, W[2], preferred_element_type=jnp.float32)
         - 2.0 * jnp.dot(s3, W[3], preferred_element_type=jnp.float32))
    pre = acc + dis[...] * m + bias[...]
    row = i * BLK + lax.broadcasted_iota(jnp.int32, (BLK, D), 0)
    pre = jnp.where(row < N, pre, 0.0)
    pre_ref[...] = pre

    @pl.when(i == 0)
    def _():
        st_ref[...] = jnp.zeros_like(st_ref)

    st_ref[0:1, :] += jnp.sum(pre, axis=0, keepdims=True)
    st_ref[1:2, :] += jnp.sum(pre * pre, axis=0, keepdims=True)


_row_spec = pl.BlockSpec((BLK, D), lambda i: (i, 0))

_k_pre = pl.pallas_call(
    _pre_body,
    grid=(NPAD // BLK,),
    in_specs=[_row_spec] * 8 + [
        pl.BlockSpec((4, D, D), lambda i: (0, 0, 0)),
        pl.BlockSpec((1, D), lambda i: (0, 0)),
    ],
    out_specs=(_row_spec, pl.BlockSpec((2, D), lambda i: (0, 0))),
    out_shape=(jax.ShapeDtypeStruct((NPAD, D), jnp.float32),
               jax.ShapeDtypeStruct((2, D), jnp.float32)),
)


def _gn(pre, st, gw, gb, gms):
    mean = st[0:1, :] / N
    ex2 = st[1:2, :] / N
    ms = gms[...]
    var = ex2 - 2.0 * ms * mean * mean + ms * ms * mean * mean
    return gw[...] * (pre[...] - ms * mean) * lax.rsqrt(var + EPS) + gb[...]


def _apply_mid_body(pre, st, dis, gw, gb, gms, h_ref, u0_ref):
    g = _gn(pre, st, gw, gb, gms)
    h = jnp.where(g >= 0, g, 0.1 * g)
    row = lax.broadcasted_iota(jnp.int32, (NPAD, D), 0)
    h = jnp.where(row < N, h, 0.0)
    h_ref[...] = h
    u0_ref[...] = dis[...] * h


_k_apply_mid = pl.pallas_call(
    _apply_mid_body,
    out_shape=(jax.ShapeDtypeStruct((NPAD, D), jnp.float32),
               jax.ShapeDtypeStruct((NPAD, D), jnp.float32)),
)


def _apply_last_body(pre, st, feat, gw, gb, gms, lw, lb, o_ref):
    g = _gn(pre, st, gw, gb, gms)
    h = jnp.maximum(feat[...] + g, 0.0)
    row = lax.broadcasted_iota(jnp.int32, (NPAD, D), 0)
    h = jnp.where(row < N, h, 0.0)
    pooled = jnp.maximum(jnp.sum(h, axis=0, keepdims=True) / N, 0.0)
    logits = jnp.dot(pooled, lw[...], preferred_element_type=jnp.float32)
    logits = logits + lb[...]
    mx = jnp.max(logits, axis=-1, keepdims=True)
    e = jnp.exp(logits - mx)
    o_ref[...] = e / jnp.sum(e, axis=-1, keepdims=True)


_k_apply_last = pl.pallas_call(
    _apply_last_body,
    out_shape=jax.ShapeDtypeStruct((1, OUT), jnp.float32),
)


def _branch(featp, srcp, dstp, dis, dsq, W, bc, gw, gb, gms, lw, lb):
    h = featp
    u0 = _k_scale(dis, featp)
    for i in range(4):
        a1 = _s2(u0, srcp, dstp)
        u1 = _k_c1(a1[0], a1[1], dsq)
        a2 = _s2(u1, srcp, dstp)
        u2 = _k_c2(a2[0], a2[1], dsq, u0)
        a3 = _s2(u2, srcp, dstp)
        pre, st = _k_pre(h, a1[0], a1[1], a2[0], a2[1], a3[0], a3[1], dis,
                         W[i], bc[i][None, :])
        if i < 3:
            h, u0 = _k_apply_mid(pre, st, dis, gw[i][None, :], gb[i][None, :],
                                 gms[i][None, :])
        else:
            out = _k_apply_last(pre, st, featp, gw[3][None, :],
                                gb[3][None, :], gms[3][None, :], lw,
                                lb[None, :])
    return out[0]


def kernel(edge_index, feat, feat_1, feat_2, W1, b1, gn1_w, gn1_b, gn1_ms,
           lin1_w, lin1_b, W2, b2, gn2_w, gn2_b, gn2_ms, lin2_w, lin2_b,
           W3, b3, gn3_w, gn3_b, gn3_ms, lin3_w, lin3_b):
    src = edge_index[0]
    dst = edge_index[1]
    pad = jnp.full((EPAD - E,), NPAD - 1, jnp.int32)
    srcp = jnp.concatenate([src, pad])
    dstp = jnp.concatenate([dst, pad])
    f1 = jnp.pad(feat, ((0, NPAD - N), (0, 0)))
    f2 = jnp.pad(feat_1, ((0, NPAD - N), (0, 0)))
    f3 = jnp.pad(feat_2, ((0, NPAD - N), (0, 0)))
    deg2 = _sdeg(srcp)
    dis, dsq = _k_dis(deg2[0], deg2[1])

    out1 = _branch(f1, srcp, dstp, dis, dsq, W1, b1, gn1_w, gn1_b, gn1_ms,
                   lin1_w, lin1_b)
    out2 = _branch(f2, srcp, dstp, dis, dsq, W2, b2, gn2_w, gn2_b, gn2_ms,
                   lin2_w, lin2_b)
    out3 = _branch(f3, srcp, dstp, dis, dsq, W3, b3, gn3_w, gn3_b, gn3_ms,
                   lin3_w, lin3_b)
    return (out1, out2, out3)
